# baseline (speedup 1.0000x reference)
import functools

import jax
import jax.numpy as jnp
from jax import lax
from jax.experimental import pallas as pl
from jax.experimental.pallas import tpu as pltpu
from jax.experimental.pallas import tpu_sc as plsc

N = 10000
E = 320000
D = 128
NC = 2
NS = 16
LANES = 16

ENC_B = 80
ENC_CHUNKS = (N + ENC_B - 1) // ENC_B
ENC_TRIPS = (ENC_CHUNKS + NS - 1) // NS


def _iota16():
    return lax.broadcasted_iota(jnp.int32, (LANES,), 0)


ENC_DH = D // NC
ENC_TROWS = 1072


def _enc_body(xadj_hbm, tabs2_hbm, out_hbm, xidx0, xidx1, tabs_v, acc_v,
              semi0, semi1, semo):
    cid = lax.axis_index("c")
    sid = lax.axis_index("s")
    xidx = (xidx0, xidx1)
    semi = (semi0, semi1)

    pltpu.sync_copy(tabs2_hbm.at[cid], tabs_v)

    def start_xidx(t, p):
        base = pl.multiple_of((sid + t * NS) * ENC_B, ENC_B)
        for i in range(9):
            pltpu.async_copy(xadj_hbm.at[pl.ds(i * N + base, ENC_B)],
                             xidx[p].at[pl.ds(i * ENC_B, ENC_B)], semi[p])

    def wait_xidx(p):
        pltpu.make_async_copy(xadj_hbm.at[pl.ds(0, 9 * ENC_B)], xidx[p],
                              semi[p]).wait()

    def out_desc(t):
        base = pl.multiple_of(cid * N + (sid + t * NS) * ENC_B, ENC_B)
        return pltpu.make_async_copy(
            acc_v, out_hbm.at[pl.ds(base, ENC_B)], semo)

    def valid(t):
        return sid + t * NS < ENC_CHUNKS

    iota = _iota16()
    start_xidx(0, 0)

    for t in range(ENC_TRIPS):
        p = t % 2

        @pl.when(valid(t))
        def _():
            wait_xidx(p)
            if t + 1 < ENC_TRIPS:
                @pl.when(valid(t + 1))
                def _():
                    start_xidx(t + 1, 1 - p)
            if t > 0:
                out_desc(t - 1).wait()

            xv = xidx[p]

            @plsc.parallel_loop(0, ENC_B, step=1, unroll=2)
            def _(j):
                rs = [plsc.load_gather(
                    xv, [jnp.full((LANES,), i * ENC_B, jnp.int32) + j])
                    for i in range(9)]
                rh = [lax.shift_right_logical(r, 1) for r in rs]
                rl = [lax.shift_left(r & 1, 6) for r in rs]
                for k in range(ENC_DH // LANES):
                    col = iota + (k * LANES)
                    v = plsc.load_gather(tabs_v, [rh[0], rl[0] + col])
                    for i in range(1, 9):
                        v = v + plsc.load_gather(tabs_v, [rh[i], rl[i] + col])
                    acc_v[j, pl.ds(k * LANES, LANES)] = v

            base = pl.multiple_of(cid * N + (sid + t * NS) * ENC_B, ENC_B)
            pltpu.async_copy(acc_v, out_hbm.at[pl.ds(base, ENC_B)], semo)

    @pl.when(valid(ENC_TRIPS - 1))
    def _():
        out_desc(ENC_TRIPS - 1).wait()

    @pl.when(jnp.logical_not(valid(ENC_TRIPS - 1)))
    def _():
        out_desc(ENC_TRIPS - 2).wait()


@functools.partial(jax.jit, static_argnums=())
def _sc_encode(xadj, tabs2):
    mesh = plsc.VectorSubcoreMesh(
        core_axis_name="c", subcore_axis_name="s", num_cores=NC, num_subcores=NS
    )
    f = pl.kernel(
        _enc_body,
        out_type=jax.ShapeDtypeStruct((NC * N, ENC_DH), jnp.float32),
        mesh=mesh,
        compiler_params=pltpu.CompilerParams(needs_layout_passes=False),
        scratch_types=[
            pltpu.VMEM((9 * ENC_B,), jnp.int32),
            pltpu.VMEM((9 * ENC_B,), jnp.int32),
            pltpu.VMEM((ENC_TROWS // 2, D), jnp.float32),
            pltpu.VMEM((ENC_B, ENC_DH), jnp.float32),
            pltpu.SemaphoreType.DMA,
            pltpu.SemaphoreType.DMA,
            pltpu.SemaphoreType.DMA,
        ],
    )
    return f(xadj, tabs2)


CONV_B = 64
NW = NC * NS
E_CHUNKS = E // CONV_B
CONV_TRIPS = E_CHUNKS // NW
CONV_REM = E_CHUNKS - CONV_TRIPS * NW
CONV_QUADS = (CONV_TRIPS + 1 + 3) // 4
WB_CHUNKS = N // CONV_B
WB_TAIL = N - WB_CHUNKS * CONV_B
WB_TRIPS = (WB_CHUNKS + 1 + NS - 1) // NS


def _conv_body(h_hbm, src_hbm, dst_hbm, cidx_hbm, ctab_hbm, out_hbm,
               agg_s, ctab_v, idx0, idx1, idx2, cidv0, cidv1, cidv2,
               dstv0, dstv1, dstv2, hr0, hr1, hr2,
               mg0, semi0, semi1, semi2, semg0, semg1, semg2, sems0):
    cid_ax = lax.axis_index("c")
    sid = lax.axis_index("s")
    wid = sid * NC + cid_ax
    tw = CONV_TRIPS + jnp.where(wid < CONV_REM, 1, 0)

    idxr = (idx0, idx1, idx2)
    cidr = (cidv0, cidv1, cidv2)
    dstr = (dstv0, dstv1, dstv2)
    hrr = (hr0, hr1, hr2)
    semi = (semi0, semi1, semi2)
    semg = (semg0, semg1, semg2)

    iota = _iota16()

    def start_idx(i, p3):
        off = pl.multiple_of((wid + i * NW) * CONV_B, CONV_B)
        pltpu.async_copy(src_hbm.at[pl.ds(off, CONV_B)], idxr[p3], semi[p3])
        pltpu.async_copy(cidx_hbm.at[pl.ds(off, CONV_B)], cidr[p3], semi[p3])
        pltpu.async_copy(dst_hbm.at[pl.ds(off, CONV_B)], dstr[p3], semi[p3])

    def wait_idx(p3):
        pltpu.make_async_copy(src_hbm.at[pl.ds(0, CONV_B)], idxr[p3], semi[p3]).wait()
        pltpu.make_async_copy(src_hbm.at[pl.ds(0, CONV_B)], cidr[p3], semi[p3]).wait()
        pltpu.make_async_copy(src_hbm.at[pl.ds(0, CONV_B)], dstr[p3], semi[p3]).wait()

    def start_gather(p3):
        pltpu.async_copy(h_hbm.at[idxr[p3]], hrr[p3], semg[p3])

    def wait_gather(p3):
        pltpu.make_async_copy(h_hbm.at[idxr[p3]], hrr[p3], semg[p3]).wait()

    def compute(p3):
        hr = hrr[p3]
        mg = mg0
        cv = cidr[p3]

        @plsc.parallel_loop(0, CONV_B, step=1, unroll=4)
        def _(j):
            c16 = plsc.load_gather(cv, [jnp.full((LANES,), j, jnp.int32)])
            for k in range(D // LANES):
                sl = pl.ds(k * LANES, LANES)
                crow = plsc.load_gather(ctab_v, [c16, iota + (k * LANES)])
                mg[j, sl] = jnp.maximum(hr[j, sl] + crow, 0.0)

    start_idx(0, 0)
    start_idx(1, 1)
    start_idx(2, 2)

    zero = jnp.zeros((LANES,), jnp.float32)

    @plsc.parallel_loop(0, CONV_B, step=1, unroll=4)
    def _(j):
        for k in range(D // LANES):
            mg0[j, pl.ds(k * LANES, LANES)] = zero

    pltpu.sync_copy(ctab_hbm, ctab_v)

    def zero_trip(c, issue):
        @pl.when(c < WB_CHUNKS)
        def _():
            off = pl.multiple_of(c * CONV_B, CONV_B)
            d = pltpu.make_async_copy(mg0, agg_s.at[pl.ds(off, CONV_B)], sems0)
            d.start() if issue else d.wait()

        if WB_TAIL:
            @pl.when(c == WB_CHUNKS)
            def _():
                d = pltpu.make_async_copy(
                    mg0.at[pl.ds(0, WB_TAIL)],
                    agg_s.at[pl.ds(WB_CHUNKS * CONV_B, WB_TAIL)], sems0)
                d.start() if issue else d.wait()

    for t in range(WB_TRIPS):
        zero_trip(sid + NS * t, True)
    wait_idx(0)
    start_gather(0)
    wait_idx(1)
    start_gather(1)
    for t in range(WB_TRIPS):
        zero_trip(sid + NS * t, False)

    plsc.subcore_barrier()

    def tri_body(tt, _):
        for q in range(3):
            i = tt * 3 + q
            g3 = q

            @pl.when(i < tw)
            def _():
                wait_gather(g3)

                @pl.when(i + 2 < tw)
                def _():
                    wait_idx((g3 + 2) % 3)
                    start_gather((g3 + 2) % 3)

                compute(g3)
                pltpu.sync_copy(mg0, agg_s.at[dstr[g3]], add=True)

                @pl.when(i + 3 < tw)
                def _():
                    start_idx(i + 3, g3)
        return 0

    lax.fori_loop(0, (CONV_TRIPS + 1 + 2) // 3, tri_body, 0)

    plsc.subcore_barrier()

    def wb_trip(c, issue):
        @pl.when(c < WB_CHUNKS)
        def _():
            off = pl.multiple_of(c * CONV_B, CONV_B)
            d = pltpu.make_async_copy(agg_s.at[pl.ds(off, CONV_B)],
                                      out_hbm.at[cid_ax, pl.ds(off, CONV_B)],
                                      sems0)
            d.start() if issue else d.wait()

        if WB_TAIL:
            @pl.when(c == WB_CHUNKS)
            def _():
                d = pltpu.make_async_copy(
                    agg_s.at[pl.ds(WB_CHUNKS * CONV_B, WB_TAIL)],
                    out_hbm.at[cid_ax, pl.ds(WB_CHUNKS * CONV_B, WB_TAIL)],
                    sems0)
                d.start() if issue else d.wait()

    for t in range(WB_TRIPS):
        wb_trip(sid + NS * t, True)
    for t in range(WB_TRIPS):
        wb_trip(sid + NS * t, False)


def _sc_conv(h, src, dst, cidx, ctab):
    mesh = plsc.VectorSubcoreMesh(
        core_axis_name="c", subcore_axis_name="s", num_cores=NC, num_subcores=NS
    )
    f = pl.kernel(
        _conv_body,
        out_type=jax.ShapeDtypeStruct((NC, N, D), jnp.float32),
        mesh=mesh,
        compiler_params=pltpu.CompilerParams(needs_layout_passes=False),
        scratch_types=(
            [pltpu.VMEM_SHARED((N, D), jnp.float32),
             pltpu.VMEM((72, D), jnp.float32)]
            + [pltpu.VMEM((CONV_B,), jnp.int32)] * 3
            + [pltpu.VMEM((CONV_B,), jnp.int32)] * 3
            + [pltpu.VMEM((CONV_B,), jnp.int32)] * 3
            + [pltpu.VMEM((CONV_B, D), jnp.float32)] * 3
            + [pltpu.VMEM((CONV_B, D), jnp.float32)] * 1
            + [pltpu.SemaphoreType.DMA] * 7
        ),
    )
    return f(h, src, dst, cidx, ctab)


def _mlp_body(relu_out, h_ref, agg_ref, eps_ref, w1_ref, b1_ref, g1_ref,
              bb1_ref, w2_ref, b2_ref, g2_ref, bb2_ref, out_ref):
    h = h_ref[...]
    pre = (1.0 + eps_ref[0, 0]) * h + agg_ref[0] + agg_ref[1]
    z = jnp.dot(pre, w1_ref[...], preferred_element_type=jnp.float32)
    z = z + b1_ref[...]
    mu = jnp.mean(z, axis=0, keepdims=True)
    var = jnp.mean((z - mu) ** 2, axis=0, keepdims=True)
    z = (z - mu) * lax.rsqrt(var + 1e-5) * g1_ref[...] + bb1_ref[...]
    z = jnp.maximum(z, 0.0)
    z = jnp.dot(z, w2_ref[...], preferred_element_type=jnp.float32)
    z = z + b2_ref[...]
    mu2 = jnp.mean(z, axis=0, keepdims=True)
    var2 = jnp.mean((z - mu2) ** 2, axis=0, keepdims=True)
    z = (z - mu2) * lax.rsqrt(var2 + 1e-5) * g2_ref[...] + bb2_ref[...]
    if relu_out:
        z = jnp.maximum(z, 0.0)
    out_ref[...] = z


def _tc_mlp(h, agg, eps_l, w1, b1, g1, bb1, w2, b2, g2, bb2, relu_out):
    return pl.pallas_call(
        functools.partial(_mlp_body, relu_out),
        out_shape=jax.ShapeDtypeStruct((N, D), jnp.float32),
    )(h, agg, eps_l, w1, b1, g1, bb1, w2, b2, g2, bb2)


def kernel(x, edge_index, edge_attr, atom_tables, bond_tables, eps,
           W1, b1, bn1_g, bn1_b, W2, b2, bn_g, bn_b):
    L = W1.shape[0]
    x = x.astype(jnp.int32)
    xadj = (x.T + (jnp.arange(9, dtype=jnp.int32)
                   * atom_tables.shape[1])[:, None]).reshape(-1)
    tabs_flat = atom_tables.reshape(-1, D)
    tabs_pad = jnp.concatenate(
        [tabs_flat,
         jnp.zeros((ENC_TROWS - tabs_flat.shape[0], D), jnp.float32)], axis=0)
    tabs2 = jnp.stack([tabs_pad[:, :ENC_DH].reshape(ENC_TROWS // 2, D),
                       tabs_pad[:, ENC_DH:].reshape(ENC_TROWS // 2, D)])
    src = edge_index[0]
    dst = edge_index[1]
    cidx = edge_attr[:, 0] * 12 + edge_attr[:, 1] * 2 + edge_attr[:, 2]

    hp = _sc_encode(xadj, tabs2)
    h = jnp.concatenate([hp[:N], hp[N:]], axis=1)
    for l in range(L):
        bt = bond_tables[l]
        ctab = (bt[0, :6][:, None, None, :] + bt[1, :6][None, :, None, :]
                + bt[2, :2][None, None, :, :]).reshape(72, D)
        agg = _sc_conv(h, src, dst, cidx, ctab)
        h = _tc_mlp(h, agg, eps[l].reshape(1, 1), W1[l], b1[l].reshape(1, D),
                    bn1_g[l].reshape(1, D), bn1_b[l].reshape(1, D), W2[l],
                    b2[l].reshape(1, D), bn_g[l].reshape(1, D),
                    bn_b[l].reshape(1, D), l < L - 1)
    return h


# speedup vs baseline: 1.2963x; 1.2963x over previous
import functools

import jax
import jax.numpy as jnp
from jax import lax
from jax.experimental import pallas as pl
from jax.experimental.pallas import tpu as pltpu
from jax.experimental.pallas import tpu_sc as plsc

N = 10000
E = 320000
D = 128
NC = 2
NS = 16
LANES = 16

ENC_B = 80
ENC_CHUNKS = (N + ENC_B - 1) // ENC_B
ENC_TRIPS = (ENC_CHUNKS + NS - 1) // NS


def _iota16():
    return lax.broadcasted_iota(jnp.int32, (LANES,), 0)


ENC_DH = D // NC
ENC_TROWS = 1072


def _enc_body(xadj_hbm, tabs2_hbm, out_hbm, xidx0, xidx1, tabs_v, acc_v,
              semi0, semi1, semo):
    cid = lax.axis_index("c")
    sid = lax.axis_index("s")
    xidx = (xidx0, xidx1)
    semi = (semi0, semi1)

    pltpu.sync_copy(tabs2_hbm.at[cid], tabs_v)

    def start_xidx(t, p):
        base = pl.multiple_of((sid + t * NS) * ENC_B, ENC_B)
        for i in range(9):
            pltpu.async_copy(xadj_hbm.at[pl.ds(i * N + base, ENC_B)],
                             xidx[p].at[pl.ds(i * ENC_B, ENC_B)], semi[p])

    def wait_xidx(p):
        pltpu.make_async_copy(xadj_hbm.at[pl.ds(0, 9 * ENC_B)], xidx[p],
                              semi[p]).wait()

    def out_desc(t):
        base = pl.multiple_of(cid * N + (sid + t * NS) * ENC_B, ENC_B)
        return pltpu.make_async_copy(
            acc_v, out_hbm.at[pl.ds(base, ENC_B)], semo)

    def valid(t):
        return sid + t * NS < ENC_CHUNKS

    iota = _iota16()
    start_xidx(0, 0)

    for t in range(ENC_TRIPS):
        p = t % 2

        @pl.when(valid(t))
        def _():
            wait_xidx(p)
            if t + 1 < ENC_TRIPS:
                @pl.when(valid(t + 1))
                def _():
                    start_xidx(t + 1, 1 - p)
            if t > 0:
                out_desc(t - 1).wait()

            xv = xidx[p]

            @plsc.parallel_loop(0, ENC_B, step=1, unroll=2)
            def _(j):
                rs = [plsc.load_gather(
                    xv, [jnp.full((LANES,), i * ENC_B, jnp.int32) + j])
                    for i in range(9)]
                rh = [lax.shift_right_logical(r, 1) for r in rs]
                rl = [lax.shift_left(r & 1, 6) for r in rs]
                for k in range(ENC_DH // LANES):
                    col = iota + (k * LANES)
                    v = plsc.load_gather(tabs_v, [rh[0], rl[0] + col])
                    for i in range(1, 9):
                        v = v + plsc.load_gather(tabs_v, [rh[i], rl[i] + col])
                    acc_v[j, pl.ds(k * LANES, LANES)] = v

            base = pl.multiple_of(cid * N + (sid + t * NS) * ENC_B, ENC_B)
            pltpu.async_copy(acc_v, out_hbm.at[pl.ds(base, ENC_B)], semo)

    @pl.when(valid(ENC_TRIPS - 1))
    def _():
        out_desc(ENC_TRIPS - 1).wait()

    @pl.when(jnp.logical_not(valid(ENC_TRIPS - 1)))
    def _():
        out_desc(ENC_TRIPS - 2).wait()


@functools.partial(jax.jit, static_argnums=())
def _sc_encode(xadj, tabs2):
    mesh = plsc.VectorSubcoreMesh(
        core_axis_name="c", subcore_axis_name="s", num_cores=NC, num_subcores=NS
    )
    f = pl.kernel(
        _enc_body,
        out_type=jax.ShapeDtypeStruct((NC * N, ENC_DH), jnp.float32),
        mesh=mesh,
        compiler_params=pltpu.CompilerParams(needs_layout_passes=False),
        scratch_types=[
            pltpu.VMEM((9 * ENC_B,), jnp.int32),
            pltpu.VMEM((9 * ENC_B,), jnp.int32),
            pltpu.VMEM((ENC_TROWS // 2, D), jnp.float32),
            pltpu.VMEM((ENC_B, ENC_DH), jnp.float32),
            pltpu.SemaphoreType.DMA,
            pltpu.SemaphoreType.DMA,
            pltpu.SemaphoreType.DMA,
        ],
    )
    return f(xadj, tabs2)


CONV_B = 64
NW = NC * NS
E_CHUNKS = E // CONV_B
CONV_TRIPS = E_CHUNKS // NW
CONV_REM = E_CHUNKS - CONV_TRIPS * NW
CONV_QUADS = (CONV_TRIPS + 1 + 3) // 4
WB_CHUNKS = N // CONV_B
WB_TAIL = N - WB_CHUNKS * CONV_B
WB_TRIPS = (WB_CHUNKS + 1 + NS - 1) // NS


def _conv_body(h_hbm, src_hbm, dst_hbm, cidx_hbm, ctab_hbm, out_hbm,
               agg_s, ctab_v, idx0, idx1, cidv0, cidv1, cidv2, cidv3,
               dstv0, dstv1, dstv2, dstv3, hr0, hr1, mg0, mg1,
               semi0, semi1, semg0, semg1, sems0, sems1):
    cid_ax = lax.axis_index("c")
    sid = lax.axis_index("s")
    wid = sid * NC + cid_ax
    tw = CONV_TRIPS + jnp.where(wid < CONV_REM, 1, 0)

    idxr = (idx0, idx1)
    cidr = (cidv0, cidv1, cidv2, cidv3)
    dstr = (dstv0, dstv1, dstv2, dstv3)
    hrr = (hr0, hr1)
    msgr = (mg0, mg1)
    semi = (semi0, semi1)
    semg = (semg0, semg1)
    sems = (sems0, sems1)

    iota = _iota16()

    def start_idx(i, p2, p4):
        off = pl.multiple_of((wid + i * NW) * CONV_B, CONV_B)
        pltpu.async_copy(src_hbm.at[pl.ds(off, CONV_B)], idxr[p2], semi[p2])
        pltpu.async_copy(cidx_hbm.at[pl.ds(off, CONV_B)], cidr[p4], semi[p2])
        pltpu.async_copy(dst_hbm.at[pl.ds(off, CONV_B)], dstr[p4], semi[p2])

    def wait_idx(p2, p4):
        pltpu.make_async_copy(src_hbm.at[pl.ds(0, CONV_B)], idxr[p2], semi[p2]).wait()
        pltpu.make_async_copy(src_hbm.at[pl.ds(0, CONV_B)], cidr[p4], semi[p2]).wait()
        pltpu.make_async_copy(src_hbm.at[pl.ds(0, CONV_B)], dstr[p4], semi[p2]).wait()

    def start_gather(p2):
        pltpu.async_copy(h_hbm.at[idxr[p2]], hrr[p2], semg[p2])

    def wait_gather(p2):
        pltpu.make_async_copy(h_hbm.at[idxr[p2]], hrr[p2], semg[p2]).wait()

    def start_scatter(p2, p4):
        pltpu.async_copy(msgr[p2], agg_s.at[dstr[p4]], sems[p2], add=True)

    def wait_scatter(p2, p4):
        pltpu.make_async_copy(msgr[p2], agg_s.at[dstr[p4]], sems[p2]).wait()

    def compute(p2, p4):
        hr = hrr[p2]
        mg = msgr[p2]
        cv = cidr[p4]

        @plsc.parallel_loop(0, CONV_B, step=1, unroll=4)
        def _(j):
            c16 = plsc.load_gather(cv, [jnp.full((LANES,), j, jnp.int32)])
            for k in range(D // LANES):
                sl = pl.ds(k * LANES, LANES)
                crow = plsc.load_gather(ctab_v, [c16, iota + (k * LANES)])
                mg[j, sl] = jnp.maximum(hr[j, sl] + crow, 0.0)

    start_idx(0, 0, 0)
    start_idx(1, 1, 1)

    zero = jnp.zeros((LANES,), jnp.float32)

    @plsc.parallel_loop(0, CONV_B, step=1, unroll=4)
    def _(j):
        for k in range(D // LANES):
            mg0[j, pl.ds(k * LANES, LANES)] = zero

    pltpu.sync_copy(ctab_hbm, ctab_v)

    def zero_trip(c, issue):
        @pl.when(c < WB_CHUNKS)
        def _():
            off = pl.multiple_of(c * CONV_B, CONV_B)
            d = pltpu.make_async_copy(mg0, agg_s.at[pl.ds(off, CONV_B)], sems0)
            d.start() if issue else d.wait()

        if WB_TAIL:
            @pl.when(c == WB_CHUNKS)
            def _():
                d = pltpu.make_async_copy(
                    mg0.at[pl.ds(0, WB_TAIL)],
                    agg_s.at[pl.ds(WB_CHUNKS * CONV_B, WB_TAIL)], sems0)
                d.start() if issue else d.wait()

    for t in range(WB_TRIPS):
        zero_trip(sid + NS * t, True)
    wait_idx(0, 0)
    start_gather(0)
    for t in range(WB_TRIPS):
        zero_trip(sid + NS * t, False)

    plsc.subcore_barrier()

    def quad_body(tt, _):
        for q in range(4):
            i = tt * 4 + q
            hb = q % 2
            cs = q

            @pl.when(i < tw)
            def _():
                wait_gather(hb)

                @pl.when(i >= 2)
                def _():
                    wait_scatter(hb, (cs + 2) % 4)

                @pl.when(i + 1 < tw)
                def _():
                    wait_idx(1 - hb, (cs + 1) % 4)
                    start_gather(1 - hb)

                @pl.when(i + 2 < tw)
                def _():
                    start_idx(i + 2, hb, (cs + 2) % 4)

                compute(hb, cs)
                start_scatter(hb, cs)
        return 0

    lax.fori_loop(0, CONV_QUADS, quad_body, 0)

    wait_scatter(0, 0)
    wait_scatter(1, 1)

    plsc.subcore_barrier()

    def wb_trip(c, issue):
        @pl.when(c < WB_CHUNKS)
        def _():
            off = pl.multiple_of(c * CONV_B, CONV_B)
            d = pltpu.make_async_copy(agg_s.at[pl.ds(off, CONV_B)],
                                      out_hbm.at[cid_ax, pl.ds(off, CONV_B)],
                                      sems0)
            d.start() if issue else d.wait()

        if WB_TAIL:
            @pl.when(c == WB_CHUNKS)
            def _():
                d = pltpu.make_async_copy(
                    agg_s.at[pl.ds(WB_CHUNKS * CONV_B, WB_TAIL)],
                    out_hbm.at[cid_ax, pl.ds(WB_CHUNKS * CONV_B, WB_TAIL)],
                    sems0)
                d.start() if issue else d.wait()

    for t in range(WB_TRIPS):
        wb_trip(sid + NS * t, True)
    for t in range(WB_TRIPS):
        wb_trip(sid + NS * t, False)


def _sc_conv(h, src, dst, cidx, ctab):
    mesh = plsc.VectorSubcoreMesh(
        core_axis_name="c", subcore_axis_name="s", num_cores=NC, num_subcores=NS
    )
    f = pl.kernel(
        _conv_body,
        out_type=jax.ShapeDtypeStruct((NC, N, D), jnp.float32),
        mesh=mesh,
        compiler_params=pltpu.CompilerParams(needs_layout_passes=False),
        scratch_types=(
            [pltpu.VMEM_SHARED((N, D), jnp.float32),
             pltpu.VMEM((72, D), jnp.float32)]
            + [pltpu.VMEM((CONV_B,), jnp.int32)] * 2
            + [pltpu.VMEM((CONV_B,), jnp.int32)] * 4
            + [pltpu.VMEM((CONV_B,), jnp.int32)] * 4
            + [pltpu.VMEM((CONV_B, D), jnp.float32)] * 2
            + [pltpu.VMEM((CONV_B, D), jnp.float32)] * 2
            + [pltpu.SemaphoreType.DMA] * 6
        ),
    )
    return f(h, src, dst, cidx, ctab)


def _mlp_body(relu_out, h_ref, agg_ref, eps_ref, w1_ref, b1_ref, g1_ref,
              bb1_ref, w2_ref, b2_ref, g2_ref, bb2_ref, out_ref):
    h = h_ref[...]
    pre = (1.0 + eps_ref[0, 0]) * h + agg_ref[0] + agg_ref[1]
    z = jnp.dot(pre, w1_ref[...], preferred_element_type=jnp.float32)
    z = z + b1_ref[...]
    mu = jnp.mean(z, axis=0, keepdims=True)
    var = jnp.mean((z - mu) ** 2, axis=0, keepdims=True)
    z = (z - mu) * lax.rsqrt(var + 1e-5) * g1_ref[...] + bb1_ref[...]
    z = jnp.maximum(z, 0.0)
    z = jnp.dot(z, w2_ref[...], preferred_element_type=jnp.float32)
    z = z + b2_ref[...]
    mu2 = jnp.mean(z, axis=0, keepdims=True)
    var2 = jnp.mean((z - mu2) ** 2, axis=0, keepdims=True)
    z = (z - mu2) * lax.rsqrt(var2 + 1e-5) * g2_ref[...] + bb2_ref[...]
    if relu_out:
        z = jnp.maximum(z, 0.0)
    out_ref[...] = z


def _tc_mlp(h, agg, eps_l, w1, b1, g1, bb1, w2, b2, g2, bb2, relu_out):
    return pl.pallas_call(
        functools.partial(_mlp_body, relu_out),
        out_shape=jax.ShapeDtypeStruct((N, D), jnp.float32),
    )(h, agg, eps_l, w1, b1, g1, bb1, w2, b2, g2, bb2)


def kernel(x, edge_index, edge_attr, atom_tables, bond_tables, eps,
           W1, b1, bn1_g, bn1_b, W2, b2, bn_g, bn_b):
    L = W1.shape[0]
    x = x.astype(jnp.int32)
    xadj = (x.T + (jnp.arange(9, dtype=jnp.int32)
                   * atom_tables.shape[1])[:, None]).reshape(-1)
    tabs_flat = atom_tables.reshape(-1, D)
    tabs_pad = jnp.concatenate(
        [tabs_flat,
         jnp.zeros((ENC_TROWS - tabs_flat.shape[0], D), jnp.float32)], axis=0)
    tabs2 = jnp.stack([tabs_pad[:, :ENC_DH].reshape(ENC_TROWS // 2, D),
                       tabs_pad[:, ENC_DH:].reshape(ENC_TROWS // 2, D)])
    src = edge_index[0]
    dst = edge_index[1]
    cidx = edge_attr[:, 0] * 12 + edge_attr[:, 1] * 2 + edge_attr[:, 2]

    hp = _sc_encode(xadj, tabs2)
    h = jnp.concatenate([hp[:N], hp[N:]], axis=1)
    for l in range(L):
        bt = bond_tables[l]
        ctab = (bt[0, :6][:, None, None, :] + bt[1, :6][None, :, None, :]
                + bt[2, :2][None, None, :, :]).reshape(72, D)
        agg = _sc_conv(h, src, dst, cidx, ctab)
        h = _tc_mlp(h, agg, eps[l].reshape(1, 1), W1[l], b1[l].reshape(1, D),
                    bn1_g[l].reshape(1, D), bn1_b[l].reshape(1, D), W2[l],
                    b2[l].reshape(1, D), bn_g[l].reshape(1, D),
                    bn_b[l].reshape(1, D), l < L - 1)
    return h
